# baseline (speedup 1.0000x reference)
import functools

import jax
import jax.numpy as jnp
from jax import lax
from jax.experimental import pallas as pl
from jax.experimental.pallas import tpu as pltpu
from jax.experimental.pallas import tpu_sc as plsc

N_NODES = 10000
N_EDGES = 320000
D_IN = 128
D_OUT = 128

NC = 2
NS = 16
NW = NC * NS
CHUNK = 128
NCHUNK = 80
EPW = NCHUNK * CHUNK
E_PAD = NW * EPW
ROWS_PT = 624
ROWS_LAST = N_NODES - (NS - 1) * ROWS_PT
LANES = 16
VPR = D_OUT // LANES


def _mm_body(x_ref, w_ref, o_ref):
    o_ref[...] = jnp.dot(x_ref[...], w_ref[...],
                         preferred_element_type=jnp.float32)


def _matmul(x, W):
    grid = 10
    rows = N_NODES // grid
    return pl.pallas_call(
        _mm_body,
        grid=(grid,),
        in_specs=[
            pl.BlockSpec((rows, D_IN), lambda i: (i, 0)),
            pl.BlockSpec((D_IN, D_OUT), lambda i: (0, 0)),
        ],
        out_specs=pl.BlockSpec((rows, D_OUT), lambda i: (i, 0)),
        out_shape=jax.ShapeDtypeStruct((N_NODES, D_OUT), jnp.float32),
    )(x, W)


_mesh = plsc.VectorSubcoreMesh(core_axis_name="c", subcore_axis_name="s")


@functools.partial(
    pl.kernel,
    out_type=jax.ShapeDtypeStruct((NC, N_NODES, D_OUT), jnp.float32),
    mesh=_mesh,
    scratch_types=[
        pltpu.VMEM((3, CHUNK), jnp.int32),
        pltpu.VMEM((3, CHUNK), jnp.int32),
        pltpu.VMEM((CHUNK, D_OUT), jnp.float32),
        pltpu.VMEM((CHUNK, D_OUT), jnp.float32),
        pltpu.VMEM_SHARED((N_NODES, D_OUT), jnp.float32),
        pltpu.SemaphoreType.DMA,
        pltpu.SemaphoreType.DMA,
        pltpu.SemaphoreType.DMA,
        pltpu.SemaphoreType.DMA,
    ],
    compiler_params=pltpu.CompilerParams(needs_layout_passes=False),
)
def _sc_edges(pk_hbm, h_hbm, out_hbm,
              idx0, idx1, rows0, rows1, acc_sh,
              gat0, gat1, scat0, scat1):
    cid = lax.axis_index("c")
    sid = lax.axis_index("s")
    wid = sid * NC + cid

    zvec = jnp.zeros((LANES,), jnp.float32)

    def _zero_row(r, _):
        for j in range(VPR):
            rows0[r, pl.ds(j * LANES, LANES)] = zvec
        return 0

    lax.fori_loop(0, CHUNK, _zero_row, 0)
    row_base = pl.multiple_of(sid * ROWS_PT, 8)
    nfull = ROWS_PT // CHUNK
    rem = ROWS_PT - nfull * CHUNK
    rem_last = ROWS_LAST - nfull * CHUNK
    for z in range(nfull):
        pltpu.sync_copy(rows0,
                        acc_sh.at[pl.ds(row_base + z * CHUNK, CHUNK)])

    @pl.when(sid < NS - 1)
    def _zero_tail():
        pltpu.sync_copy(rows0.at[pl.ds(0, rem)],
                        acc_sh.at[pl.ds(row_base + nfull * CHUNK, rem)])

    @pl.when(sid == NS - 1)
    def _zero_tail_last():
        pltpu.sync_copy(rows0.at[pl.ds(0, rem_last)],
                        acc_sh.at[pl.ds((NS - 1) * ROWS_PT + nfull * CHUNK,
                                        rem_last)])

    def _scale(ib, rb):
        def _group(g, _):
            wv = plsc.bitcast(ib[2, pl.ds(g * LANES, LANES)], jnp.float32)
            for t in range(LANES):
                e = g * LANES + t
                w = wv[t]
                for j in range(VPR):
                    sl = pl.ds(j * LANES, LANES)
                    rb[e, sl] = rb[e, sl] * w
            return 0

        lax.fori_loop(0, CHUNK // LANES, _group, 0)

    plsc.subcore_barrier()

    def _pair(t, _):
        i0 = 2 * t
        i1 = 2 * t + 1
        pltpu.sync_copy(pk_hbm.at[wid, i0], idx0)
        g0 = pltpu.async_copy(h_hbm.at[idx0.at[0]], rows0, gat0)
        pltpu.sync_copy(pk_hbm.at[wid, i1], idx1)
        g0.wait()
        g1 = pltpu.async_copy(h_hbm.at[idx1.at[0]], rows1, gat1)
        _scale(idx0, rows0)
        s0 = pltpu.async_copy(rows0, acc_sh.at[idx0.at[1]], scat0, add=True)
        g1.wait()
        _scale(idx1, rows1)
        s0.wait()
        s1 = pltpu.async_copy(rows1, acc_sh.at[idx1.at[1]], scat1, add=True)
        s1.wait()
        return 0

    lax.fori_loop(0, NCHUNK // 2, _pair, 0)
    plsc.subcore_barrier()

    @pl.when(sid < NS - 1)
    def _wb_main():
        pltpu.sync_copy(acc_sh.at[pl.ds(row_base, ROWS_PT)],
                        out_hbm.at[cid, pl.ds(row_base, ROWS_PT)])

    @pl.when(sid == NS - 1)
    def _wb_last():
        last = (NS - 1) * ROWS_PT
        pltpu.sync_copy(acc_sh.at[pl.ds(last, ROWS_LAST)],
                        out_hbm.at[cid, pl.ds(last, ROWS_LAST)])


def _add_body(a_ref, b_ref, o_ref):
    o_ref[...] = a_ref[...] + b_ref[...]


def _combine(p0, p1):
    grid = 10
    rows = N_NODES // grid
    return pl.pallas_call(
        _add_body,
        grid=(grid,),
        in_specs=[
            pl.BlockSpec((rows, D_OUT), lambda i: (i, 0)),
            pl.BlockSpec((rows, D_OUT), lambda i: (i, 0)),
        ],
        out_specs=pl.BlockSpec((rows, D_OUT), lambda i: (i, 0)),
        out_shape=jax.ShapeDtypeStruct((N_NODES, D_OUT), jnp.float32),
    )(p0, p1)


def kernel(x, edge_index, edge_weight, W):
    edge_index = edge_index.astype(jnp.int32)
    pad = E_PAD - N_EDGES
    src = jnp.concatenate(
        [edge_index[0], jnp.zeros((pad,), jnp.int32)]).reshape(
            NW, NCHUNK, CHUNK)
    dst = jnp.concatenate(
        [edge_index[1], jnp.zeros((pad,), jnp.int32)]).reshape(
            NW, NCHUNK, CHUNK)
    wbits = jax.lax.bitcast_convert_type(
        jnp.concatenate([edge_weight, jnp.zeros((pad,), jnp.float32)]),
        jnp.int32).reshape(NW, NCHUNK, CHUNK)
    packed = jnp.stack([src, dst, wbits], axis=2)
    h = _matmul(x, W)
    partials = _sc_edges(packed, h)
    return _combine(partials[0], partials[1])


# speedup vs baseline: 1.3093x; 1.3093x over previous
import functools

import jax
import jax.numpy as jnp
from jax import lax
from jax.experimental import pallas as pl
from jax.experimental.pallas import tpu as pltpu
from jax.experimental.pallas import tpu_sc as plsc

N_NODES = 10000
N_EDGES = 320000
D_IN = 128
D_OUT = 128

NC = 2
NS = 16
NW = NC * NS
EPW = N_EDGES // NW
CHUNK = 80
NCHUNK = EPW // CHUNK
ROWS_PT = 624
ROWS_LAST = N_NODES - (NS - 1) * ROWS_PT
ZROWS = 208
LANES = 16
VPR = D_OUT // LANES


def _mm_body(x_ref, w_ref, o_ref):
    o_ref[...] = jnp.dot(x_ref[...], w_ref[...],
                         preferred_element_type=jnp.float32)


def _matmul(x, W):
    grid = 10
    rows = N_NODES // grid
    return pl.pallas_call(
        _mm_body,
        grid=(grid,),
        in_specs=[
            pl.BlockSpec((rows, D_IN), lambda i: (i, 0)),
            pl.BlockSpec((D_IN, D_OUT), lambda i: (0, 0)),
        ],
        out_specs=pl.BlockSpec((rows, D_OUT), lambda i: (i, 0)),
        out_shape=jax.ShapeDtypeStruct((N_NODES, D_OUT), jnp.float32),
    )(x, W)


_mesh = plsc.VectorSubcoreMesh(core_axis_name="c", subcore_axis_name="s")


@functools.partial(
    pl.kernel,
    out_type=jax.ShapeDtypeStruct((NC, N_NODES, D_OUT), jnp.float32),
    mesh=_mesh,
    scratch_types=[
        pltpu.VMEM((CHUNK,), jnp.int32),
        pltpu.VMEM((CHUNK,), jnp.int32),
        pltpu.VMEM((CHUNK,), jnp.float32),
        pltpu.VMEM((CHUNK, D_OUT), jnp.float32),
        pltpu.VMEM((ZROWS, D_OUT), jnp.float32),
        pltpu.VMEM_SHARED((N_NODES, D_OUT), jnp.float32),
        pltpu.SemaphoreType.DMA,
    ],
)
def _sc_edges(src_hbm, dst_hbm, w_hbm, h_hbm, out_hbm,
              src_v, dst_v, w_v, rows_v, zero_v, acc_sh, sem):
    cid = lax.axis_index("c")
    sid = lax.axis_index("s")
    wid = sid * NC + cid

    zvec = jnp.zeros((LANES,), jnp.float32)

    def _zero_row(r, _):
        for j in range(VPR):
            zero_v[r, pl.ds(j * LANES, LANES)] = zvec
        return 0

    lax.fori_loop(0, ZROWS, _zero_row, 0)
    row_base = pl.multiple_of(sid * ROWS_PT, 8)
    for z in range(ROWS_PT // ZROWS):
        pltpu.sync_copy(zero_v,
                        acc_sh.at[pl.ds(row_base + z * ZROWS, ZROWS)])

    @pl.when(sid == NS - 1)
    def _zero_tail():
        pltpu.sync_copy(
            zero_v.at[pl.ds(0, ROWS_LAST - 3 * ZROWS)],
            acc_sh.at[pl.ds((NS - 1) * ROWS_PT + 3 * ZROWS,
                            ROWS_LAST - 3 * ZROWS)])

    plsc.subcore_barrier()

    def _chunk(i, _):
        base = pl.multiple_of(wid * EPW + i * CHUNK, CHUNK)
        pltpu.sync_copy(src_hbm.at[pl.ds(base, CHUNK)], src_v)
        pltpu.sync_copy(dst_hbm.at[pl.ds(base, CHUNK)], dst_v)
        pltpu.sync_copy(w_hbm.at[pl.ds(base, CHUNK)], w_v)
        pltpu.async_copy(h_hbm.at[src_v], rows_v, sem).wait()

        def _group(g, _):
            wv = w_v[pl.ds(g * LANES, LANES)]
            for t in range(LANES):
                e = g * LANES + t
                w = wv[t]
                for j in range(VPR):
                    sl = pl.ds(j * LANES, LANES)
                    rows_v[e, sl] = rows_v[e, sl] * w
            return 0

        lax.fori_loop(0, CHUNK // LANES, _group, 0)
        pltpu.sync_copy(rows_v, acc_sh.at[dst_v], add=True)
        return 0

    lax.fori_loop(0, NCHUNK, _chunk, 0)
    plsc.subcore_barrier()

    @pl.when(sid < NS - 1)
    def _wb_main():
        pltpu.sync_copy(acc_sh.at[pl.ds(row_base, ROWS_PT)],
                        out_hbm.at[cid, pl.ds(row_base, ROWS_PT)])

    @pl.when(sid == NS - 1)
    def _wb_last():
        last = (NS - 1) * ROWS_PT
        pltpu.sync_copy(acc_sh.at[pl.ds(last, ROWS_LAST)],
                        out_hbm.at[cid, pl.ds(last, ROWS_LAST)])


def _add_body(a_ref, b_ref, o_ref):
    o_ref[...] = a_ref[...] + b_ref[...]


def _combine(p0, p1):
    grid = 10
    rows = N_NODES // grid
    return pl.pallas_call(
        _add_body,
        grid=(grid,),
        in_specs=[
            pl.BlockSpec((rows, D_OUT), lambda i: (i, 0)),
            pl.BlockSpec((rows, D_OUT), lambda i: (i, 0)),
        ],
        out_specs=pl.BlockSpec((rows, D_OUT), lambda i: (i, 0)),
        out_shape=jax.ShapeDtypeStruct((N_NODES, D_OUT), jnp.float32),
    )(p0, p1)


def kernel(x, edge_index, edge_weight, W):
    edge_index = edge_index.astype(jnp.int32)
    src = edge_index[0]
    dst = edge_index[1]
    h = _matmul(x, W)
    partials = _sc_edges(src, dst, edge_weight, h)
    return _combine(partials[0], partials[1])
